# Initial kernel scaffold; baseline (speedup 1.0000x reference)
import jax
import jax.numpy as jnp
from jax.experimental import pallas as pl


def kernel(x, pos_emb):
    raise NotImplementedError("write your pallas kernel here")


# speedup vs baseline: 1.6675x; 1.6675x over previous
import jax
import jax.numpy as jnp
from jax.experimental import pallas as pl


B, S, D = 4, 8192, 1024
S_BLK = 1024


def _add_body(x_ref, pos_ref, o_ref):
    o_ref[...] = x_ref[...] + pos_ref[...][None, :, :]


def kernel(x, pos_emb):
    b, s, d = x.shape
    n_s = s // S_BLK
    pos = pos_emb[:s]
    return pl.pallas_call(
        _add_body,
        grid=(n_s, b),
        in_specs=[
            pl.BlockSpec((1, S_BLK, d), lambda i_s, i_b: (i_b, i_s, 0)),
            pl.BlockSpec((S_BLK, d), lambda i_s, i_b: (i_s, 0)),
        ],
        out_specs=pl.BlockSpec((1, S_BLK, d), lambda i_s, i_b: (i_b, i_s, 0)),
        out_shape=jax.ShapeDtypeStruct((b, s, d), x.dtype),
    )(x, pos)


# speedup vs baseline: 1.6701x; 1.0016x over previous
import jax
import jax.numpy as jnp
from jax.experimental import pallas as pl
from jax.experimental.pallas import tpu as pltpu


B, S, D = 4, 8192, 1024
S_BLK = 1024


def _add_body(x_ref, pos_ref, o_ref):
    o_ref[...] = x_ref[...] + pos_ref[...][None, :, :]


def kernel(x, pos_emb):
    b, s, d = x.shape
    n_s = s // S_BLK
    pos = pos_emb[:s]
    return pl.pallas_call(
        _add_body,
        grid=(n_s, b),
        in_specs=[
            pl.BlockSpec((1, S_BLK, d), lambda i_s, i_b: (i_b, i_s, 0)),
            pl.BlockSpec((S_BLK, d), lambda i_s, i_b: (i_s, 0)),
        ],
        out_specs=pl.BlockSpec((1, S_BLK, d), lambda i_s, i_b: (i_b, i_s, 0)),
        out_shape=jax.ShapeDtypeStruct((b, s, d), x.dtype),
        compiler_params=pltpu.CompilerParams(
            dimension_semantics=("parallel", "arbitrary"),
        ),
    )(x, pos)


# speedup vs baseline: 1.7393x; 1.0414x over previous
import jax
import jax.numpy as jnp
from jax.experimental import pallas as pl
from jax.experimental.pallas import tpu as pltpu


B, S, D = 4, 8192, 1024
S_BLK = 2048


def _add_body(x_ref, pos_ref, o_ref):
    o_ref[...] = x_ref[...] + pos_ref[...][None, :, :]


def kernel(x, pos_emb):
    b, s, d = x.shape
    n_s = s // S_BLK
    pos = pos_emb[:s]
    return pl.pallas_call(
        _add_body,
        grid=(n_s, b),
        in_specs=[
            pl.BlockSpec((1, S_BLK, d), lambda i_s, i_b: (i_b, i_s, 0)),
            pl.BlockSpec((S_BLK, d), lambda i_s, i_b: (i_s, 0)),
        ],
        out_specs=pl.BlockSpec((1, S_BLK, d), lambda i_s, i_b: (i_b, i_s, 0)),
        out_shape=jax.ShapeDtypeStruct((b, s, d), x.dtype),
        compiler_params=pltpu.CompilerParams(
            dimension_semantics=("parallel", "arbitrary"),
        ),
    )(x, pos)


# speedup vs baseline: 1.7402x; 1.0005x over previous
import jax
import jax.numpy as jnp
from jax.experimental import pallas as pl
from jax.experimental.pallas import tpu as pltpu


B, S, D = 4, 8192, 1024
S_BLK = 2048


def _add_body(x_ref, pos_ref, o_ref):
    o_ref[...] = x_ref[...] + pos_ref[...][None, :, :]


def kernel(x, pos_emb):
    b, s, d = x.shape
    n_s = s // S_BLK
    pos = pos_emb[:s]
    return pl.pallas_call(
        _add_body,
        grid=(n_s, b),
        in_specs=[
            pl.BlockSpec((1, S_BLK, d), lambda i_s, i_b: (i_b, i_s, 0)),
            pl.BlockSpec((S_BLK, d), lambda i_s, i_b: (i_s, 0)),
        ],
        out_specs=pl.BlockSpec((1, S_BLK, d), lambda i_s, i_b: (i_b, i_s, 0)),
        out_shape=jax.ShapeDtypeStruct((b, s, d), x.dtype),
        compiler_params=pltpu.CompilerParams(
            dimension_semantics=("parallel", "arbitrary"),
            vmem_limit_bytes=128 * 1024 * 1024,
        ),
    )(x, pos)
